# baseline (speedup 1.0000x reference)
import functools

import jax
import jax.numpy as jnp
from jax import lax
from jax.experimental import pallas as pl
from jax.experimental.pallas import tpu as pltpu
from jax.experimental.pallas import tpu_sc as plsc

NC = 2
NS = 16
NW = NC * NS

EMB = 64
BLK = 128
L = 16


TP = 129


def _transpose_rows_to_tiles(src, dst, rows, cols):
    lanes = lax.iota(jnp.int32, L)
    G = 8

    def col_chunk(ci, carry):
        del carry
        c0 = ci * L
        cl = c0 + lanes
        hi = lax.shift_right_logical(cl, 3)
        lo = lax.bitwise_and(cl, 7)

        def store(r, v):
            plsc.store_scatter(dst, [hi, lo, jnp.full((L,), r, jnp.int32)], v)

        vs = [src[r, pl.ds(c0, L)] for r in range(G)]
        for r0 in range(G, rows, G):
            for k in range(G):
                store(r0 - G + k, vs[k])
                vs[k] = src[r0 + k, pl.ds(c0, L)]
        for k in range(G):
            store(rows - G + k, vs[k])
        return 0

    lax.fori_loop(0, cols // L, col_chunk, 0)


NBUF = 4

VT = 7813
VPW = 245
ROWS_PAD = VT * BLK
FP = 137


def _fmt_body(tt_hbm, tab_hbm, va0, va1, vb0, vb1, isem0, isem1,
              osem0, osem1):
    wid = lax.axis_index("s") * NC + lax.axis_index("c")
    start = wid * VPW
    stop = jnp.minimum(start + VPW, VT)

    va = (va0, va1)
    vb = (vb0, vb1)
    isems = (isem0, isem1)
    osems = (osem0, osem1)
    lanes = lax.iota(jnp.int32, L)

    def in_descr(vt, b):
        return pltpu.make_async_copy(
            tt_hbm.at[:, pl.ds(vt * BLK, BLK)], va[b], isems[b])

    def out_descr(vt, b):
        return pltpu.make_async_copy(
            vb[b].at[:, pl.ds(0, 2 * EMB)],
            tab_hbm.at[pl.ds(vt * EMB, EMB)], osems[b])

    def transpose_block(b):
        src, dst = va[b], vb[b]
        G = 8

        def col_chunk(ci, carry):
            del carry
            vl0 = ci * L
            rows_v = vl0 + lanes
            hi = lax.shift_right_logical(rows_v, 1)
            par = lax.bitwise_and(rows_v, 1) * EMB

            def store(e, v):
                plsc.store_scatter(dst, [hi, par + e], v)

            vs = [src[e, pl.ds(vl0, L)] for e in range(G)]
            for e0 in range(G, EMB, G):
                for k in range(G):
                    store(e0 - G + k, vs[k])
                    vs[k] = src[e0 + k, pl.ds(vl0, L)]
            for k in range(G):
                store(EMB - G + k, vs[k])
            return 0

        lax.fori_loop(0, BLK // L, col_chunk, 0)

    @pl.when(start < VT)
    def _():
        in_descr(start, 0).start()

    @pl.when(start + 1 < VT)
    def _():
        in_descr(start + 1, 1).start()

    def loop_body(i, carry):
        del carry
        for b in range(2):
            vt = start + 2 * i + b

            @pl.when(vt < stop)
            def _():
                @pl.when(vt - 2 >= start)
                def _():
                    out_descr(vt - 2, b).wait()

                in_descr(vt, b).wait()
                transpose_block(b)
                out_descr(vt, b).start()

                @pl.when(vt + 2 < stop)
                def _():
                    in_descr(vt + 2, b).start()

        return 0

    lax.fori_loop(0, (VPW + 1) // 2, loop_body, 0)

    for b in range(2):
        last = stop - 2 + ((stop - start) + b) % 2

        @pl.when(last >= start)
        def _():
            out_descr(last, b).wait()


def _format_table(table):
    tt = jnp.transpose(table)
    mesh = plsc.VectorSubcoreMesh(core_axis_name="c", subcore_axis_name="s")
    return pl.kernel(
        _fmt_body,
        out_type=jax.ShapeDtypeStruct((ROWS_PAD // 2, 2 * EMB), jnp.float32),
        mesh=mesh,
        compiler_params=pltpu.CompilerParams(use_tc_tiling_on_sc=True,
                                             needs_layout_passes=False,
                                             disable_bounds_checks=True),
        scratch_types=[
            pltpu.VMEM((EMB, BLK), jnp.float32),
            pltpu.VMEM((EMB, BLK), jnp.float32),
            pltpu.VMEM((EMB, FP), jnp.float32),
            pltpu.VMEM((EMB, FP), jnp.float32),
            pltpu.SemaphoreType.DMA,
            pltpu.SemaphoreType.DMA,
            pltpu.SemaphoreType.DMA,
            pltpu.SemaphoreType.DMA,
        ],
    )(tt)


def _emb_body(idx_hbm, table_hbm, out_hbm, idxa, idxt, *bufs_and_sems, seq):
    wid = lax.axis_index("s") * NC + lax.axis_index("c")
    base = wid * BLK

    pltpu.sync_copy(idx_hbm.at[pl.ds(base, BLK)], idxa)
    lanes = lax.iota(jnp.int32, L)
    for s in range(seq):
        for r0 in range(0, BLK, L):
            v = plsc.load_gather(idxa, [lanes + r0, jnp.full((L,), s, jnp.int32)])
            idxt[s, pl.ds(r0, L)] = v

    g = bufs_and_sems[:NBUF]
    t = bufs_and_sems[NBUF:2 * NBUF]
    gsems = bufs_and_sems[2 * NBUF:3 * NBUF]
    osems = bufs_and_sems[3 * NBUF:4 * NBUF]

    def gather_descr(s, b):
        return pltpu.make_async_copy(table_hbm.at[idxt.at[s]], g[b], gsems[b])

    def out_descr(s, b):
        return pltpu.make_async_copy(t[b].at[:, :, pl.ds(0, BLK)],
                                     out_hbm.at[s, :, wid], osems[b])

    for b in range(NBUF):
        gather_descr(b, b).start()

    def loop_body(i, carry):
        del carry
        for b in range(NBUF):
            s = NBUF * i + b

            @pl.when(s >= NBUF)
            def _():
                out_descr(s - NBUF, b).wait()

            gather_descr(s, b).wait()
            _transpose_rows_to_tiles(g[b], t[b], BLK, EMB)
            out_descr(s, b).start()

            @pl.when(s + NBUF < seq)
            def _():
                gather_descr(s + NBUF, b).start()

        return 0

    lax.fori_loop(0, seq // NBUF, loop_body, 0)

    for b in range(NBUF):
        out_descr(seq - NBUF + b, b).wait()


def _emb_lookup(idx_bm, table, batch, seq):
    mesh = plsc.VectorSubcoreMesh(core_axis_name="c", subcore_axis_name="s")
    body = functools.partial(_emb_body, seq=seq)
    return pl.kernel(
        body,
        out_type=jax.ShapeDtypeStruct((seq, EMB // 8, batch // BLK, 8, BLK),
                                      jnp.float32),
        mesh=mesh,
        compiler_params=pltpu.CompilerParams(use_tc_tiling_on_sc=False,
                                             needs_layout_passes=False),
        scratch_types=(
            [pltpu.VMEM((BLK, seq), jnp.int32),
             pltpu.VMEM((seq, BLK), jnp.int32)]
            + [pltpu.VMEM((BLK, EMB), jnp.float32)] * NBUF
            + [pltpu.VMEM((EMB // 8, 8, TP), jnp.float32)] * NBUF
            + [pltpu.SemaphoreType.DMA] * (2 * NBUF)
        ),
    )(idx_bm, table)


def kernel(x, table):
    batch, seq = x.shape
    idx_bm = x.astype(jnp.int32)
    tab_lin = jnp.reshape(_format_table(table), (ROWS_PAD, EMB))
    out5 = _emb_lookup(idx_bm, tab_lin, batch, seq)
    out = jnp.transpose(out5, (2, 4, 0, 1, 3))
    return jnp.reshape(out, (batch, seq, EMB))


# speedup vs baseline: 1.0014x; 1.0014x over previous
import functools

import jax
import jax.numpy as jnp
from jax import lax
from jax.experimental import pallas as pl
from jax.experimental.pallas import tpu as pltpu
from jax.experimental.pallas import tpu_sc as plsc

NC = 2
NS = 16
NW = NC * NS

EMB = 64
BLK = 128
L = 16


TP = 129


def _transpose_rows_to_tiles(src, dst, rows, cols):
    lanes = lax.iota(jnp.int32, L)
    G = 8

    def col_chunk(ci, carry):
        del carry
        c0 = ci * L
        cl = c0 + lanes
        hi = lax.shift_right_logical(cl, 3)
        lo = lax.bitwise_and(cl, 7)

        def store(r, v):
            plsc.store_scatter(dst, [hi, lo, jnp.full((L,), r, jnp.int32)], v)

        vs = [src[r, pl.ds(c0, L)] for r in range(G)]
        for r0 in range(G, rows, G):
            for k in range(G):
                store(r0 - G + k, vs[k])
                vs[k] = src[r0 + k, pl.ds(c0, L)]
        for k in range(G):
            store(rows - G + k, vs[k])
        return 0

    lax.fori_loop(0, cols // L, col_chunk, 0)


NBUF = 4

VT = 7813
VPW = 245
ROWS_PAD = VT * BLK
FP = 137


ANB = 4


def _fmt_body(tt_hbm, tab_hbm, *bufs_and_sems):
    wid = lax.axis_index("s") * NC + lax.axis_index("c")
    start = wid * VPW
    stop = jnp.minimum(start + VPW, VT)

    va = bufs_and_sems[:ANB]
    vb = bufs_and_sems[ANB:2 * ANB]
    isems = bufs_and_sems[2 * ANB:3 * ANB]
    osems = bufs_and_sems[3 * ANB:4 * ANB]
    lanes = lax.iota(jnp.int32, L)

    def in_descr(vt, b):
        return pltpu.make_async_copy(
            tt_hbm.at[:, pl.ds(vt * BLK, BLK)], va[b], isems[b])

    def out_descr(vt, b):
        return pltpu.make_async_copy(
            vb[b].at[:, pl.ds(0, 2 * EMB)],
            tab_hbm.at[pl.ds(vt * EMB, EMB)], osems[b])

    def transpose_block(b):
        src, dst = va[b], vb[b]
        G = 8

        def col_chunk(ci, carry):
            del carry
            vl0 = ci * L
            rows_v = vl0 + lanes
            hi = lax.shift_right_logical(rows_v, 1)
            par = lax.bitwise_and(rows_v, 1) * EMB

            def store(e, v):
                plsc.store_scatter(dst, [hi, par + e], v)

            vs = [src[e, pl.ds(vl0, L)] for e in range(G)]
            for e0 in range(G, EMB, G):
                for k in range(G):
                    store(e0 - G + k, vs[k])
                    vs[k] = src[e0 + k, pl.ds(vl0, L)]
            for k in range(G):
                store(EMB - G + k, vs[k])
            return 0

        lax.fori_loop(0, BLK // L, col_chunk, 0)

    for b in range(ANB):
        @pl.when(start + b < VT)
        def _(b=b):
            in_descr(start + b, b).start()

    def loop_body(i, carry):
        del carry
        for b in range(ANB):
            vt = start + ANB * i + b

            @pl.when(vt < stop)
            def _():
                @pl.when(vt - ANB >= start)
                def _():
                    out_descr(vt - ANB, b).wait()

                in_descr(vt, b).wait()
                transpose_block(b)
                out_descr(vt, b).start()

                @pl.when(vt + ANB < stop)
                def _():
                    in_descr(vt + ANB, b).start()

        return 0

    lax.fori_loop(0, (VPW + ANB - 1) // ANB, loop_body, 0)

    for b in range(ANB):
        cnt = stop - start
        last = start + ((cnt - 1 - b) // ANB) * ANB + b

        @pl.when((last >= start) & (last < stop))
        def _(last=last):
            out_descr(last, b).wait()


def _format_table(table):
    tt = jnp.transpose(table)
    mesh = plsc.VectorSubcoreMesh(core_axis_name="c", subcore_axis_name="s")
    return pl.kernel(
        _fmt_body,
        out_type=jax.ShapeDtypeStruct((ROWS_PAD // 2, 2 * EMB), jnp.float32),
        mesh=mesh,
        compiler_params=pltpu.CompilerParams(use_tc_tiling_on_sc=True,
                                             needs_layout_passes=False,
                                             disable_bounds_checks=True),
        scratch_types=(
            [pltpu.VMEM((EMB, BLK), jnp.float32)] * ANB
            + [pltpu.VMEM((EMB, FP), jnp.float32)] * ANB
            + [pltpu.SemaphoreType.DMA] * (2 * ANB)
        ),
    )(tt)


def _emb_body(idx_hbm, table_hbm, out_hbm, idxa, idxt, *bufs_and_sems, seq):
    wid = lax.axis_index("s") * NC + lax.axis_index("c")
    base = wid * BLK

    pltpu.sync_copy(idx_hbm.at[pl.ds(base, BLK)], idxa)
    lanes = lax.iota(jnp.int32, L)
    for s in range(seq):
        for r0 in range(0, BLK, L):
            v = plsc.load_gather(idxa, [lanes + r0, jnp.full((L,), s, jnp.int32)])
            idxt[s, pl.ds(r0, L)] = v

    g = bufs_and_sems[:NBUF]
    t = bufs_and_sems[NBUF:2 * NBUF]
    gsems = bufs_and_sems[2 * NBUF:3 * NBUF]
    osems = bufs_and_sems[3 * NBUF:4 * NBUF]

    def gather_descr(s, b):
        return pltpu.make_async_copy(table_hbm.at[idxt.at[s]], g[b], gsems[b])

    def out_descr(s, b):
        return pltpu.make_async_copy(t[b].at[:, :, pl.ds(0, BLK)],
                                     out_hbm.at[s, :, wid], osems[b])

    for b in range(NBUF):
        gather_descr(b, b).start()

    def loop_body(i, carry):
        del carry
        for b in range(NBUF):
            s = NBUF * i + b

            @pl.when(s >= NBUF)
            def _():
                out_descr(s - NBUF, b).wait()

            gather_descr(s, b).wait()
            _transpose_rows_to_tiles(g[b], t[b], BLK, EMB)
            out_descr(s, b).start()

            @pl.when(s + NBUF < seq)
            def _():
                gather_descr(s + NBUF, b).start()

        return 0

    lax.fori_loop(0, seq // NBUF, loop_body, 0)

    for b in range(NBUF):
        out_descr(seq - NBUF + b, b).wait()


def _emb_lookup(idx_bm, table, batch, seq):
    mesh = plsc.VectorSubcoreMesh(core_axis_name="c", subcore_axis_name="s")
    body = functools.partial(_emb_body, seq=seq)
    return pl.kernel(
        body,
        out_type=jax.ShapeDtypeStruct((seq, EMB // 8, batch // BLK, 8, BLK),
                                      jnp.float32),
        mesh=mesh,
        compiler_params=pltpu.CompilerParams(use_tc_tiling_on_sc=False,
                                             needs_layout_passes=False),
        scratch_types=(
            [pltpu.VMEM((BLK, seq), jnp.int32),
             pltpu.VMEM((seq, BLK), jnp.int32)]
            + [pltpu.VMEM((BLK, EMB), jnp.float32)] * NBUF
            + [pltpu.VMEM((EMB // 8, 8, TP), jnp.float32)] * NBUF
            + [pltpu.SemaphoreType.DMA] * (2 * NBUF)
        ),
    )(idx_bm, table)


def kernel(x, table):
    batch, seq = x.shape
    idx_bm = x.astype(jnp.int32)
    tab_lin = jnp.reshape(_format_table(table), (ROWS_PAD, EMB))
    out5 = _emb_lookup(idx_bm, tab_lin, batch, seq)
    out = jnp.transpose(out5, (2, 4, 0, 1, 3))
    return jnp.reshape(out, (batch, seq, EMB))


# speedup vs baseline: 1.3701x; 1.3683x over previous
import functools

import jax
import jax.numpy as jnp
from jax import lax
from jax.experimental import pallas as pl
from jax.experimental.pallas import tpu as pltpu
from jax.experimental.pallas import tpu_sc as plsc

NC = 2
NS = 16
NW = NC * NS

EMB = 64
BLK = 128
L = 16


TP = 129


def _transpose_rows_to_tiles(src, dst, rows, cols):
    lanes = lax.iota(jnp.int32, L)
    G = 8

    def col_chunk(ci, carry):
        del carry
        c0 = ci * L
        cl = c0 + lanes
        hi = lax.shift_right_logical(cl, 3)
        lo = lax.bitwise_and(cl, 7)

        def store(r, v):
            plsc.store_scatter(dst, [hi, lo, jnp.full((L,), r, jnp.int32)], v)

        vs = [src[r, pl.ds(c0, L)] for r in range(G)]
        for r0 in range(G, rows, G):
            for k in range(G):
                store(r0 - G + k, vs[k])
                vs[k] = src[r0 + k, pl.ds(c0, L)]
        for k in range(G):
            store(rows - G + k, vs[k])
        return 0

    lax.fori_loop(0, cols // L, col_chunk, 0)


NBUF = 4


def _emb_body(idx_hbm, table_hbm, out_hbm, idxa, idxt, *bufs_and_sems, seq):
    wid = lax.axis_index("s") * NC + lax.axis_index("c")
    base = wid * BLK

    pltpu.sync_copy(idx_hbm.at[pl.ds(base, BLK)], idxa)
    lanes = lax.iota(jnp.int32, L)
    for s in range(seq):
        for r0 in range(0, BLK, L):
            v = plsc.load_gather(idxa, [lanes + r0, jnp.full((L,), s, jnp.int32)])
            idxt[s, pl.ds(r0, L)] = v

    g = bufs_and_sems[:NBUF]
    t = bufs_and_sems[NBUF:2 * NBUF]
    gsems = bufs_and_sems[2 * NBUF:3 * NBUF]
    osems = bufs_and_sems[3 * NBUF:4 * NBUF]

    def gather_descr(s, b):
        return pltpu.make_async_copy(table_hbm.at[idxt.at[s]], g[b], gsems[b])

    def out_descr(s, b):
        return pltpu.make_async_copy(t[b].at[:, :, pl.ds(0, BLK)],
                                     out_hbm.at[s, :, wid], osems[b])

    for b in range(NBUF):
        gather_descr(b, b).start()

    def loop_body(i, carry):
        del carry
        for b in range(NBUF):
            s = NBUF * i + b

            @pl.when(s >= NBUF)
            def _():
                out_descr(s - NBUF, b).wait()

            gather_descr(s, b).wait()
            _transpose_rows_to_tiles(g[b], t[b], BLK, EMB)
            out_descr(s, b).start()

            @pl.when(s + NBUF < seq)
            def _():
                gather_descr(s + NBUF, b).start()

        return 0

    lax.fori_loop(0, seq // NBUF, loop_body, 0)

    for b in range(NBUF):
        out_descr(seq - NBUF + b, b).wait()


def _emb_lookup(idx_bm, table, batch, seq):
    mesh = plsc.VectorSubcoreMesh(core_axis_name="c", subcore_axis_name="s")
    body = functools.partial(_emb_body, seq=seq)
    return pl.kernel(
        body,
        out_type=jax.ShapeDtypeStruct((seq, EMB // 8, batch // BLK, 8, BLK),
                                      jnp.float32),
        mesh=mesh,
        compiler_params=pltpu.CompilerParams(use_tc_tiling_on_sc=False,
                                             needs_layout_passes=False),
        scratch_types=(
            [pltpu.VMEM((BLK, seq), jnp.int32),
             pltpu.VMEM((seq, BLK), jnp.int32)]
            + [pltpu.VMEM((BLK, EMB), jnp.float32)] * NBUF
            + [pltpu.VMEM((EMB // 8, 8, TP), jnp.float32)] * NBUF
            + [pltpu.SemaphoreType.DMA] * (2 * NBUF)
        ),
    )(idx_bm, table)


def kernel(x, table):
    batch, seq = x.shape
    idx_bm = x.astype(jnp.int32)
    out5 = _emb_lookup(idx_bm, table, batch, seq)
    out = jnp.transpose(out5, (2, 4, 0, 1, 3))
    return jnp.reshape(out, (batch, seq, EMB))
